# baseline (speedup 1.0000x reference)
import functools

import jax
import jax.numpy as jnp
from jax import lax
from jax.experimental import pallas as pl
from jax.experimental.pallas import tpu as pltpu, tpu_sc as plsc

ROWS = 4096
COLS = 26
VOCAB = 1000
PLANE = COLS * VOCAB
TOTAL = ROWS * PLANE
NUM_WORKERS = 32
PLANES_PER_WORKER = ROWS // NUM_WORKERS
POS_PER_WORKER = PLANES_PER_WORKER * COLS
L = 16
K = 128
CHUNKS = POS_PER_WORKER // K
MSET_F = 1_331_200
MSET_N = TOTAL // MSET_F


def _memset_body(o_ref, zbuf, sem):
    zbuf[...] = jnp.zeros((MSET_F,), jnp.float32)

    def fire(i, carry):
        pltpu.async_copy(zbuf, o_ref.at[pl.ds(i * MSET_F, MSET_F)], sem)
        return carry

    lax.fori_loop(0, MSET_N, fire, 0)

    def drain(i, carry):
        pltpu.make_async_copy(zbuf, o_ref.at[pl.ds(0, MSET_F)], sem).wait()
        return carry

    lax.fori_loop(0, MSET_N, drain, 0)


_memset = functools.partial(
    pl.kernel,
    mesh=pltpu.create_tensorcore_mesh("core"),
    scratch_types=[
        pltpu.VMEM((MSET_F,), jnp.float32),
        pltpu.SemaphoreType.DMA,
    ],
)(_memset_body)


def _scatter_body(x_hbm, out_ref, xs_v, idx_v, ones_v, sem):
    wid = lax.axis_index("c") * 16 + lax.axis_index("s")
    base = wid * PLANES_PER_WORKER

    pltpu.sync_copy(x_hbm.at[pl.ds(base, PLANES_PER_WORKER)], xs_v)

    iota = lax.iota(jnp.int32, L)
    for m in range(K // L):
        ones_v[pl.ds(m * L, L)] = jnp.full((L,), 1.0, jnp.float32)

    def fill_idx(c, carry):
        for m in range(K // L):
            r = c * K + m * L + iota
            poff = r // COLS
            j = r - poff * COLS
            cols = plsc.load_gather(xs_v, [poff, j])
            idx_v[c, pl.ds(m * L, L)] = (
                (base + poff) * PLANE + j * VOCAB + cols)
        return carry

    lax.fori_loop(0, CHUNKS, fill_idx, 0)

    def fire(c, carry):
        pltpu.async_copy(ones_v, out_ref.at[idx_v.at[c]], sem)
        return carry

    lax.fori_loop(0, CHUNKS, fire, 0)

    def drain(c, carry):
        pltpu.make_async_copy(ones_v, out_ref.at[idx_v.at[0]], sem).wait()
        return carry

    lax.fori_loop(0, CHUNKS, drain, 0)


_scatter = functools.partial(
    pl.kernel,
    mesh=plsc.VectorSubcoreMesh(core_axis_name="c", subcore_axis_name="s"),
    compiler_params=pltpu.CompilerParams(
        use_tc_tiling_on_sc=False, needs_layout_passes=False),
    scratch_types=[
        pltpu.VMEM((PLANES_PER_WORKER, COLS), jnp.int32),
        pltpu.VMEM((CHUNKS, K), jnp.int32),
        pltpu.VMEM((K,), jnp.float32),
        pltpu.SemaphoreType.DMA,
    ],
)(_scatter_body)


def kernel(x):
    out = jax.empty_ref(jax.ShapeDtypeStruct((TOTAL,), jnp.float32))
    _memset(out)
    _scatter(x, out)
    return out[...].reshape(ROWS, COLS, VOCAB)


# speedup vs baseline: 1.0001x; 1.0001x over previous
import functools

import jax
import jax.numpy as jnp
from jax import lax
from jax.experimental import pallas as pl
from jax.experimental.pallas import tpu as pltpu, tpu_sc as plsc

ROWS = 4096
COLS = 26
VOCAB = 1000
PLANE = COLS * VOCAB
TOTAL = ROWS * PLANE
NUM_WORKERS = 32
PLANES_PER_WORKER = ROWS // NUM_WORKERS
POS_PER_WORKER = PLANES_PER_WORKER * COLS
L = 16
K = 128
CHUNKS = POS_PER_WORKER // K
MSET_F = 1_331_200
MSET_N = TOTAL // MSET_F


def _memset_body(o_ref, zbuf, sem):
    zbuf[...] = jnp.zeros((MSET_F,), jnp.float32)

    def fire(i, carry):
        pltpu.async_copy(zbuf, o_ref.at[pl.ds(i * MSET_F, MSET_F)], sem)
        return carry

    lax.fori_loop(0, MSET_N, fire, 0)

    def drain(i, carry):
        pltpu.make_async_copy(zbuf, o_ref.at[pl.ds(0, MSET_F)], sem).wait()
        return carry

    lax.fori_loop(0, MSET_N, drain, 0)


_memset = functools.partial(
    pl.kernel,
    mesh=pltpu.create_tensorcore_mesh("core"),
    scratch_types=[
        pltpu.VMEM((MSET_F,), jnp.float32),
        pltpu.SemaphoreType.DMA,
    ],
)(_memset_body)


def _scatter_body(x_hbm, out_ref, xs_v, idx_v, ones_v, sem):
    wid = lax.axis_index("c") * 16 + lax.axis_index("s")
    base = wid * PLANES_PER_WORKER

    pltpu.sync_copy(x_hbm.at[pl.ds(base, PLANES_PER_WORKER)], xs_v)

    iota = lax.iota(jnp.int32, L)
    for m in range(K // L):
        ones_v[pl.ds(m * L, L)] = jnp.full((L,), 1.0, jnp.float32)

    def fill_idx(c, carry):
        for m in range(K // L):
            r = c * K + m * L + iota
            poff = r // COLS
            j = r - poff * COLS
            cols = plsc.load_gather(xs_v, [poff, j])
            idx_v[c, pl.ds(m * L, L)] = (
                (base + poff) * PLANE + j * VOCAB + cols)
        return carry

    lax.fori_loop(0, CHUNKS, fill_idx, 0)

    def fire(c, carry):
        pltpu.async_copy(ones_v, out_ref.at[idx_v.at[c]], sem)
        return carry

    lax.fori_loop(0, CHUNKS, fire, 0)

    def drain(c, carry):
        pltpu.make_async_copy(ones_v, out_ref.at[idx_v.at[0]], sem).wait()
        return carry

    lax.fori_loop(0, CHUNKS, drain, 0)


_scatter = functools.partial(
    pl.kernel,
    mesh=plsc.VectorSubcoreMesh(core_axis_name="c", subcore_axis_name="s"),
    compiler_params=pltpu.CompilerParams(
        use_tc_tiling_on_sc=False, needs_layout_passes=False),
    scratch_types=[
        pltpu.VMEM((PLANES_PER_WORKER, COLS), jnp.int32),
        pltpu.VMEM((CHUNKS, K), jnp.int32),
        pltpu.VMEM((K,), jnp.float32),
        pltpu.SemaphoreType.DMA,
    ],
)(_scatter_body)


def kernel(x):
    out = jax.empty_ref(jax.ShapeDtypeStruct((TOTAL,), jnp.float32))
    _memset(out)
    _scatter(x, out)
    return jax.ref.freeze(out).reshape(ROWS, COLS, VOCAB)
